# baseline (speedup 1.0000x reference)
import functools

import jax
import jax.numpy as jnp
from jax import lax
from jax.experimental import pallas as pl
from jax.experimental.pallas import tpu as pltpu

N, Q, K = 10000, 128, 10
BM1 = 2000
BM2 = 400


def _stage1_body(z_ref, wtT_ref, bt_ref, lnw_ref, lnb_ref, woT_ref, bo_ref,
                 p_ref, s_ref, x_ref, colsum_ref, ztp_ref):
    step = pl.program_id(0)
    nsteps = pl.num_programs(0)

    z = z_ref[...]
    m = jnp.dot(z, wtT_ref[...], preferred_element_type=jnp.float32)
    m = m + bt_ref[...]
    mu = jnp.mean(m, axis=1, keepdims=True)
    var = jnp.mean((m - mu) * (m - mu), axis=1, keepdims=True)
    mn = (m - mu) / jnp.sqrt(var + 1e-5) * lnw_ref[...] + lnb_ref[...]
    h = jnp.where(mn >= 0, mn, 0.2 * mn)
    ol = jnp.dot(h, woT_ref[...], preferred_element_type=jnp.float32)
    ol = ol + bo_ref[...]
    olmax = jnp.max(ol, axis=1, keepdims=True)
    e = jnp.exp(ol - olmax)
    p = e / jnp.sum(e, axis=1, keepdims=True)
    p_ref[...] = p

    pmax = jnp.max(p, axis=1, keepdims=True)
    lane = lax.broadcasted_iota(jnp.int32, p.shape, 1)
    s_ref[...] = jnp.min(jnp.where(p == pmax, lane, 127), axis=1,
                         keepdims=True)

    @pl.when(step == 0)
    def _init():
        colsum_ref[...] = jnp.zeros_like(colsum_ref)
        ztp_ref[...] = jnp.zeros_like(ztp_ref)

    colsum_ref[...] += jnp.sum(p, axis=0, keepdims=True)
    ztp_ref[...] += lax.dot_general(z, p, (((0,), (0,)), ((), ())),
                                    preferred_element_type=jnp.float32)

    @pl.when(step == nsteps - 1)
    def _fin():
        cs = colsum_ref[...]
        lane1 = lax.broadcasted_iota(jnp.int32, cs.shape, 1)
        d = jnp.where(lane1 < K, 1.0 / cs + 1e-8, 0.0)
        x_ref[...] = (ztp_ref[...] * d).T


def _stage2_body(a_ref, pbf_ref, pblk_ref, at_ref, acc_ref):
    step = pl.program_id(0)
    nsteps = pl.num_programs(0)

    a_bf = a_ref[...].astype(jnp.bfloat16)
    ap = jnp.dot(a_bf, pbf_ref[...], preferred_element_type=jnp.float32)

    @pl.when(step == 0)
    def _init():
        acc_ref[...] = jnp.zeros_like(acc_ref)

    acc_ref[...] += lax.dot_general(pblk_ref[...], ap,
                                    (((0,), (0,)), ((), ())),
                                    preferred_element_type=jnp.float32)

    @pl.when(step == nsteps - 1)
    def _fin():
        at_ref[...] = acc_ref[...]


def kernel(Z, A, W_t, b_t, ln_w, ln_b, W_o, b_o):
    wtT = W_t.T
    bt = b_t.reshape(1, Q)
    lnw = ln_w.reshape(1, Q)
    lnb = ln_b.reshape(1, Q)
    woT = jnp.zeros((Q, 128), jnp.float32).at[:, :K].set(W_o.T)
    bo = jnp.full((1, 128), -1e30, jnp.float32).at[0, :K].set(b_o)

    grid1 = N // BM1
    p_pad, s2d, x_full = pl.pallas_call(
        _stage1_body,
        grid=(grid1,),
        in_specs=[
            pl.BlockSpec((BM1, Q), lambda i: (i, 0)),
            pl.BlockSpec((Q, Q), lambda i: (0, 0)),
            pl.BlockSpec((1, Q), lambda i: (0, 0)),
            pl.BlockSpec((1, Q), lambda i: (0, 0)),
            pl.BlockSpec((1, Q), lambda i: (0, 0)),
            pl.BlockSpec((Q, 128), lambda i: (0, 0)),
            pl.BlockSpec((1, 128), lambda i: (0, 0)),
        ],
        out_specs=[
            pl.BlockSpec((BM1, 128), lambda i: (i, 0)),
            pl.BlockSpec((BM1, 1), lambda i: (i, 0)),
            pl.BlockSpec((128, 128), lambda i: (0, 0)),
        ],
        out_shape=[
            jax.ShapeDtypeStruct((N, 128), jnp.float32),
            jax.ShapeDtypeStruct((N, 1), jnp.int32),
            jax.ShapeDtypeStruct((128, 128), jnp.float32),
        ],
        scratch_shapes=[
            pltpu.VMEM((1, 128), jnp.float32),
            pltpu.VMEM((128, 128), jnp.float32),
        ],
    )(Z, wtT, bt, lnw, lnb, woT, bo)

    p_bf = p_pad.astype(jnp.bfloat16)
    grid2 = N // BM2
    at_full = pl.pallas_call(
        _stage2_body,
        grid=(grid2,),
        in_specs=[
            pl.BlockSpec((BM2, N), lambda i: (i, 0)),
            pl.BlockSpec((N, 128), lambda i: (0, 0)),
            pl.BlockSpec((BM2, 128), lambda i: (i, 0)),
        ],
        out_specs=pl.BlockSpec((128, 128), lambda i: (0, 0)),
        out_shape=jax.ShapeDtypeStruct((128, 128), jnp.float32),
        scratch_shapes=[pltpu.VMEM((128, 128), jnp.float32)],
    )(A, p_bf, p_pad)

    X_tilde = x_full[:K, :]
    A_tilde = at_full[:K, :K]
    P = p_pad[:, :K]
    S = s2d[:, 0]
    return X_tilde, A_tilde, P, S


# speedup vs baseline: 1.0060x; 1.0060x over previous
import functools

import jax
import jax.numpy as jnp
from jax import lax
from jax.experimental import pallas as pl
from jax.experimental.pallas import tpu as pltpu

N, Q, K = 10000, 128, 10
BM1 = 2000
BM2 = 400


def _stage1_body(z_ref, wtT_ref, bt_ref, lnw_ref, lnb_ref, woT_ref, bo_ref,
                 p_ref, pbf_ref, s_ref, x_ref, colsum_ref, ztp_ref):
    step = pl.program_id(0)
    nsteps = pl.num_programs(0)

    z = z_ref[...]
    m = jnp.dot(z, wtT_ref[...], preferred_element_type=jnp.float32)
    m = m + bt_ref[...]
    mu = jnp.mean(m, axis=1, keepdims=True)
    var = jnp.mean((m - mu) * (m - mu), axis=1, keepdims=True)
    mn = (m - mu) / jnp.sqrt(var + 1e-5) * lnw_ref[...] + lnb_ref[...]
    h = jnp.where(mn >= 0, mn, 0.2 * mn)
    ol = jnp.dot(h, woT_ref[...], preferred_element_type=jnp.float32)
    ol = ol + bo_ref[...]
    olmax = jnp.max(ol, axis=1, keepdims=True)
    e = jnp.exp(ol - olmax)
    p = e / jnp.sum(e, axis=1, keepdims=True)
    p_ref[...] = p
    pbf_ref[...] = p.astype(jnp.bfloat16)

    pmax = jnp.max(p, axis=1, keepdims=True)
    lane = lax.broadcasted_iota(jnp.int32, p.shape, 1)
    s_ref[...] = jnp.min(jnp.where(p == pmax, lane, 127), axis=1,
                         keepdims=True)

    @pl.when(step == 0)
    def _init():
        colsum_ref[...] = jnp.zeros_like(colsum_ref)
        ztp_ref[...] = jnp.zeros_like(ztp_ref)

    colsum_ref[...] += jnp.sum(p, axis=0, keepdims=True)
    ztp_ref[...] += lax.dot_general(z, p, (((0,), (0,)), ((), ())),
                                    preferred_element_type=jnp.float32)

    @pl.when(step == nsteps - 1)
    def _fin():
        cs = colsum_ref[...]
        lane1 = lax.broadcasted_iota(jnp.int32, cs.shape, 1)
        d = jnp.where(lane1 < K, 1.0 / cs + 1e-8, 0.0)
        x_ref[...] = (ztp_ref[...] * d).T


def _stage2_body(a_ref, pbf_ref, pblk_ref, at_ref, acc_ref):
    step = pl.program_id(0)
    nsteps = pl.num_programs(0)

    a_bf = a_ref[...].astype(jnp.bfloat16)
    ap = jnp.dot(a_bf, pbf_ref[...], preferred_element_type=jnp.float32)

    @pl.when(step == 0)
    def _init():
        acc_ref[...] = jnp.zeros_like(acc_ref)

    acc_ref[...] += lax.dot_general(pblk_ref[...], ap,
                                    (((0,), (0,)), ((), ())),
                                    preferred_element_type=jnp.float32)

    @pl.when(step == nsteps - 1)
    def _fin():
        at_ref[...] = acc_ref[...]


def kernel(Z, A, W_t, b_t, ln_w, ln_b, W_o, b_o):
    wtT = W_t.T
    bt = b_t.reshape(1, Q)
    lnw = ln_w.reshape(1, Q)
    lnb = ln_b.reshape(1, Q)
    woT = jnp.zeros((Q, 128), jnp.float32).at[:, :K].set(W_o.T)
    bo = jnp.full((1, 128), -1e30, jnp.float32).at[0, :K].set(b_o)

    grid1 = N // BM1
    p_pad, p_bf, s2d, x_full = pl.pallas_call(
        _stage1_body,
        grid=(grid1,),
        in_specs=[
            pl.BlockSpec((BM1, Q), lambda i: (i, 0)),
            pl.BlockSpec((Q, Q), lambda i: (0, 0)),
            pl.BlockSpec((1, Q), lambda i: (0, 0)),
            pl.BlockSpec((1, Q), lambda i: (0, 0)),
            pl.BlockSpec((1, Q), lambda i: (0, 0)),
            pl.BlockSpec((Q, 128), lambda i: (0, 0)),
            pl.BlockSpec((1, 128), lambda i: (0, 0)),
        ],
        out_specs=[
            pl.BlockSpec((BM1, 128), lambda i: (i, 0)),
            pl.BlockSpec((BM1, 128), lambda i: (i, 0)),
            pl.BlockSpec((BM1, 1), lambda i: (i, 0)),
            pl.BlockSpec((128, 128), lambda i: (0, 0)),
        ],
        out_shape=[
            jax.ShapeDtypeStruct((N, 128), jnp.float32),
            jax.ShapeDtypeStruct((N, 128), jnp.bfloat16),
            jax.ShapeDtypeStruct((N, 1), jnp.int32),
            jax.ShapeDtypeStruct((128, 128), jnp.float32),
        ],
        scratch_shapes=[
            pltpu.VMEM((1, 128), jnp.float32),
            pltpu.VMEM((128, 128), jnp.float32),
        ],
    )(Z, wtT, bt, lnw, lnb, woT, bo)

    grid2 = N // BM2
    at_full = pl.pallas_call(
        _stage2_body,
        grid=(grid2,),
        in_specs=[
            pl.BlockSpec((BM2, N), lambda i: (i, 0)),
            pl.BlockSpec((N, 128), lambda i: (0, 0)),
            pl.BlockSpec((BM2, 128), lambda i: (i, 0)),
        ],
        out_specs=pl.BlockSpec((128, 128), lambda i: (0, 0)),
        out_shape=jax.ShapeDtypeStruct((128, 128), jnp.float32),
        scratch_shapes=[pltpu.VMEM((128, 128), jnp.float32)],
    )(A, p_bf, p_pad)

    X_tilde = x_full[:K, :]
    A_tilde = at_full[:K, :K]
    P = p_pad[:, :K]
    S = s2d[:, 0]
    return X_tilde, A_tilde, P, S


# speedup vs baseline: 1.0110x; 1.0050x over previous
import functools

import jax
import jax.numpy as jnp
from jax import lax
from jax.experimental import pallas as pl
from jax.experimental.pallas import tpu as pltpu

N, Q, K = 10000, 128, 10
BM1 = 2000
BM2 = 200
HB = N // 2 // BM2


def _stage1_body(z_ref, wtT_ref, bt_ref, lnw_ref, lnb_ref, woT_ref, bo_ref,
                 p_ref, pbf_ref, s_ref, x_ref, colsum_ref, ztp_ref):
    step = pl.program_id(0)
    nsteps = pl.num_programs(0)

    z = z_ref[...]
    m = jnp.dot(z, wtT_ref[...], preferred_element_type=jnp.float32)
    m = m + bt_ref[...]
    mu = jnp.mean(m, axis=1, keepdims=True)
    var = jnp.mean((m - mu) * (m - mu), axis=1, keepdims=True)
    mn = (m - mu) / jnp.sqrt(var + 1e-5) * lnw_ref[...] + lnb_ref[...]
    h = jnp.where(mn >= 0, mn, 0.2 * mn)
    ol = jnp.dot(h, woT_ref[...], preferred_element_type=jnp.float32)
    ol = ol + bo_ref[...]
    olmax = jnp.max(ol, axis=1, keepdims=True)
    e = jnp.exp(ol - olmax)
    p = e / jnp.sum(e, axis=1, keepdims=True)
    p_ref[...] = p
    pbf_ref[...] = p.astype(jnp.bfloat16)

    pmax = jnp.max(p, axis=1, keepdims=True)
    lane = lax.broadcasted_iota(jnp.int32, p.shape, 1)
    s_ref[...] = jnp.min(jnp.where(p == pmax, lane, 127), axis=1,
                         keepdims=True)

    @pl.when(step == 0)
    def _init():
        colsum_ref[...] = jnp.zeros_like(colsum_ref)
        ztp_ref[...] = jnp.zeros_like(ztp_ref)

    colsum_ref[...] += jnp.sum(p, axis=0, keepdims=True)
    ztp_ref[...] += lax.dot_general(z, p, (((0,), (0,)), ((), ())),
                                    preferred_element_type=jnp.float32)

    @pl.when(step == nsteps - 1)
    def _fin():
        cs = colsum_ref[...]
        lane1 = lax.broadcasted_iota(jnp.int32, cs.shape, 1)
        d = jnp.where(lane1 < K, 1.0 / cs + 1e-8, 0.0)
        x_ref[...] = (ztp_ref[...] * d).T


def _stage2_body(a1_ref, a2_ref, pbf_ref, pblk1_ref, pblk2_ref, at_ref,
                 acc_ref):
    step = pl.program_id(0)
    nsteps = pl.num_programs(0)

    pbf = pbf_ref[...]
    ap1 = jnp.dot(a1_ref[...].astype(jnp.bfloat16), pbf,
                  preferred_element_type=jnp.float32)
    ap2 = jnp.dot(a2_ref[...].astype(jnp.bfloat16), pbf,
                  preferred_element_type=jnp.float32)

    @pl.when(step == 0)
    def _init():
        acc_ref[...] = jnp.zeros_like(acc_ref)

    acc_ref[...] += (
        lax.dot_general(pblk1_ref[...], ap1, (((0,), (0,)), ((), ())),
                        preferred_element_type=jnp.float32)
        + lax.dot_general(pblk2_ref[...], ap2, (((0,), (0,)), ((), ())),
                          preferred_element_type=jnp.float32))

    @pl.when(step == nsteps - 1)
    def _fin():
        at_ref[...] = acc_ref[...]


def kernel(Z, A, W_t, b_t, ln_w, ln_b, W_o, b_o):
    wtT = W_t.T
    bt = b_t.reshape(1, Q)
    lnw = ln_w.reshape(1, Q)
    lnb = ln_b.reshape(1, Q)
    woT = jnp.zeros((Q, 128), jnp.float32).at[:, :K].set(W_o.T)
    bo = jnp.full((1, 128), -1e30, jnp.float32).at[0, :K].set(b_o)

    grid1 = N // BM1
    p_pad, p_bf, s2d, x_full = pl.pallas_call(
        _stage1_body,
        grid=(grid1,),
        in_specs=[
            pl.BlockSpec((BM1, Q), lambda i: (i, 0)),
            pl.BlockSpec((Q, Q), lambda i: (0, 0)),
            pl.BlockSpec((1, Q), lambda i: (0, 0)),
            pl.BlockSpec((1, Q), lambda i: (0, 0)),
            pl.BlockSpec((1, Q), lambda i: (0, 0)),
            pl.BlockSpec((Q, 128), lambda i: (0, 0)),
            pl.BlockSpec((1, 128), lambda i: (0, 0)),
        ],
        out_specs=[
            pl.BlockSpec((BM1, 128), lambda i: (i, 0)),
            pl.BlockSpec((BM1, 128), lambda i: (i, 0)),
            pl.BlockSpec((BM1, 1), lambda i: (i, 0)),
            pl.BlockSpec((128, 128), lambda i: (0, 0)),
        ],
        out_shape=[
            jax.ShapeDtypeStruct((N, 128), jnp.float32),
            jax.ShapeDtypeStruct((N, 128), jnp.bfloat16),
            jax.ShapeDtypeStruct((N, 1), jnp.int32),
            jax.ShapeDtypeStruct((128, 128), jnp.float32),
        ],
        scratch_shapes=[
            pltpu.VMEM((1, 128), jnp.float32),
            pltpu.VMEM((128, 128), jnp.float32),
        ],
    )(Z, wtT, bt, lnw, lnb, woT, bo)

    at_full = pl.pallas_call(
        _stage2_body,
        grid=(HB,),
        in_specs=[
            pl.BlockSpec((BM2, N), lambda i: (i, 0)),
            pl.BlockSpec((BM2, N), lambda i: (i + HB, 0)),
            pl.BlockSpec((N, 128), lambda i: (0, 0)),
            pl.BlockSpec((BM2, 128), lambda i: (i, 0)),
            pl.BlockSpec((BM2, 128), lambda i: (i + HB, 0)),
        ],
        out_specs=pl.BlockSpec((128, 128), lambda i: (0, 0)),
        out_shape=jax.ShapeDtypeStruct((128, 128), jnp.float32),
        scratch_shapes=[pltpu.VMEM((128, 128), jnp.float32)],
    )(A, A, p_bf, p_pad, p_pad)

    X_tilde = x_full[:K, :]
    A_tilde = at_full[:K, :K]
    P = p_pad[:, :K]
    S = s2d[:, 0]
    return X_tilde, A_tilde, P, S
